# baseline (speedup 1.0000x reference)
import functools

import jax
import jax.numpy as jnp
from jax.experimental import pallas as pl
from jax.experimental.pallas import tpu as pltpu
from jax.sharding import Mesh, NamedSharding, PartitionSpec as P

try:
    from jax import shard_map as _shard_map
except ImportError:
    from jax.experimental.shard_map import shard_map as _shard_map

B = 1
S = 2048
HIDDEN = 2048
NH = 16
NKV = 8
G = NH // NKV
HD = 128
EPS = 1e-6
SCALE = HD ** -0.5

BP = 256
BQ = 512
BK = 512
BO = 512
BQ2 = BQ * G


def _rope(x, cos, sin):
    x1 = x[:, : HD // 2]
    x2 = x[:, HD // 2:]
    rot = jnp.concatenate([-x2, x1], axis=1)
    return x * cos + rot * sin


def _qkv_kernel(nh, nkv, hs_ref, wq_ref, wk_ref, wv_ref, cos_ref, sin_ref,
                qw_ref, kw_ref, q_out, k_out, v_out):
    x = hs_ref[...].astype(jnp.bfloat16)
    cos = cos_ref[...]
    sin = sin_ref[...]
    qw = qw_ref[...]
    kw = kw_ref[...]

    q = jnp.dot(x, wq_ref[...], preferred_element_type=jnp.float32)
    for h in range(nh):
        qh = q[:, h * HD:(h + 1) * HD]
        var = jnp.mean(qh * qh, axis=-1, keepdims=True)
        qh = qh * (jax.lax.rsqrt(var + EPS) * SCALE) * qw
        q_out[h] = _rope(qh, cos, sin).astype(jnp.bfloat16)

    k = jnp.dot(x, wk_ref[...], preferred_element_type=jnp.float32)
    for h in range(nkv):
        kh = k[:, h * HD:(h + 1) * HD]
        var = jnp.mean(kh * kh, axis=-1, keepdims=True)
        kh = kh * jax.lax.rsqrt(var + EPS) * kw
        k_out[h] = _rope(kh, cos, sin).astype(jnp.bfloat16)

    v = jnp.dot(x, wv_ref[...], preferred_element_type=jnp.float32)
    for h in range(nkv):
        v_out[h] = v[:, h * HD:(h + 1) * HD].astype(jnp.bfloat16)


def _attn_kernel(nkv, q_ref, k_ref, v_ref, attn_ref, acc_ref, l_ref):
    i = pl.program_id(0)

    row = jax.lax.broadcasted_iota(jnp.int32, (BQ2, BK), 0)
    col = jax.lax.broadcasted_iota(jnp.int32, (BQ2, BK), 1)
    diag_mask = col <= jax.lax.rem(row, BQ)

    for p_ in range(nkv):
        q2 = q_ref[G * p_:G * p_ + G].reshape(BQ2, HD)

        l_ref[...] = jnp.zeros((BQ2, 1), jnp.float32)
        acc_ref[...] = jnp.zeros((BQ2, HD), jnp.float32)

        def body(j, _):
            kj = k_ref[p_, pl.ds(j * BK, BK), :]
            vj = v_ref[p_, pl.ds(j * BK, BK), :]
            s = jax.lax.dot_general(
                q2, kj, (((1,), (1,)), ((), ())),
                preferred_element_type=jnp.float32)
            p = jnp.exp(s)
            l_ref[...] += jnp.sum(p, axis=1, keepdims=True)
            acc_ref[...] += jnp.dot(p.astype(jnp.bfloat16), vj,
                                    preferred_element_type=jnp.float32)
            return 0

        jax.lax.fori_loop(0, i, body, 0)

        kj = k_ref[p_, pl.ds(i * BK, BK), :]
        vj = v_ref[p_, pl.ds(i * BK, BK), :]
        s = jax.lax.dot_general(
            q2, kj, (((1,), (1,)), ((), ())),
            preferred_element_type=jnp.float32)
        p = jnp.where(diag_mask, jnp.exp(s), 0.0)
        l_ref[...] += jnp.sum(p, axis=1, keepdims=True)
        acc_ref[...] += jnp.dot(p.astype(jnp.bfloat16), vj,
                                preferred_element_type=jnp.float32)

        out2 = (acc_ref[...] / l_ref[...]).astype(jnp.bfloat16)
        for g in range(G):
            h = G * p_ + g
            attn_ref[:, h * HD:(h + 1) * HD] = out2[g * BQ:(g + 1) * BQ]


def _oproj_kernel(attn_ref, wo_ref, out_ref):
    out_ref[...] = jnp.dot(attn_ref[...], wo_ref[...],
                           preferred_element_type=jnp.float32)


def _local_forward(hs, wq, wk, wv, wo, cos2, sin2, qw, kw):
    nh = wq.shape[1] // HD
    nkv = wk.shape[1] // HD
    hid_l = wo.shape[1]

    q, k, v = pl.pallas_call(
        functools.partial(_qkv_kernel, nh, nkv),
        grid=(S // BP,),
        in_specs=[
            pl.BlockSpec((BP, HIDDEN), lambda i: (i, 0)),
            pl.BlockSpec((HIDDEN, nh * HD), lambda i: (0, 0)),
            pl.BlockSpec((HIDDEN, nkv * HD), lambda i: (0, 0)),
            pl.BlockSpec((HIDDEN, nkv * HD), lambda i: (0, 0)),
            pl.BlockSpec((BP, HD), lambda i: (i, 0)),
            pl.BlockSpec((BP, HD), lambda i: (i, 0)),
            pl.BlockSpec((1, HD), lambda i: (0, 0)),
            pl.BlockSpec((1, HD), lambda i: (0, 0)),
        ],
        out_specs=[
            pl.BlockSpec((nh, BP, HD), lambda i: (0, i, 0)),
            pl.BlockSpec((nkv, BP, HD), lambda i: (0, i, 0)),
            pl.BlockSpec((nkv, BP, HD), lambda i: (0, i, 0)),
        ],
        out_shape=[
            jax.ShapeDtypeStruct((nh, S, HD), jnp.bfloat16),
            jax.ShapeDtypeStruct((nkv, S, HD), jnp.bfloat16),
            jax.ShapeDtypeStruct((nkv, S, HD), jnp.bfloat16),
        ],
    )(hs, wq, wk, wv, cos2, sin2, qw, kw)

    attn = pl.pallas_call(
        functools.partial(_attn_kernel, nkv),
        grid=(S // BQ,),
        in_specs=[
            pl.BlockSpec((nh, BQ, HD), lambda i: (0, i, 0)),
            pl.BlockSpec((nkv, S, HD), lambda i: (0, 0, 0)),
            pl.BlockSpec((nkv, S, HD), lambda i: (0, 0, 0)),
        ],
        out_specs=pl.BlockSpec((BQ, nh * HD), lambda i: (i, 0)),
        out_shape=jax.ShapeDtypeStruct((S, nh * HD), jnp.bfloat16),
        scratch_shapes=[
            pltpu.VMEM((BQ2, HD), jnp.float32),
            pltpu.VMEM((BQ2, 1), jnp.float32),
        ],
    )(q, k, v)

    attn_full = jax.lax.all_gather(attn, "x", axis=1, tiled=True)

    out = pl.pallas_call(
        _oproj_kernel,
        grid=(S // BO,),
        in_specs=[
            pl.BlockSpec((BO, NH * HD), lambda i: (i, 0)),
            pl.BlockSpec((NH * HD, hid_l), lambda i: (0, 0)),
        ],
        out_specs=pl.BlockSpec((BO, hid_l), lambda i: (i, 0)),
        out_shape=jax.ShapeDtypeStruct((S, hid_l), jnp.float32),
    )(attn_full, wo)

    return out


@jax.jit
def kernel(hidden_states, cos, sin, Wq, Wk, Wv, Wo, q_norm_w, k_norm_w):
    ndev = min(len(jax.devices()), 2)
    mesh = Mesh(jax.devices()[:ndev], ("x",))
    repl = NamedSharding(mesh, P())
    colsh = NamedSharding(mesh, P(None, "x"))

    def cstr(x, s):
        return jax.lax.with_sharding_constraint(x, s)

    hs = cstr(hidden_states.reshape(S, HIDDEN), repl)
    cos2 = cstr(cos.reshape(S, HD), repl)
    sin2 = cstr(sin.reshape(S, HD), repl)
    qw = cstr(q_norm_w.reshape(1, HD), repl)
    kw = cstr(k_norm_w.reshape(1, HD), repl)
    wq16 = cstr(Wq.astype(jnp.bfloat16), colsh)
    wk16 = cstr(Wk.astype(jnp.bfloat16), colsh)
    wv16 = cstr(Wv.astype(jnp.bfloat16), colsh)
    wo16 = cstr(Wo.astype(jnp.bfloat16), colsh)

    fwd = _shard_map(
        _local_forward,
        mesh=mesh,
        in_specs=(P(), P(None, "x"), P(None, "x"), P(None, "x"),
                  P(None, "x"), P(), P(), P(), P()),
        out_specs=P(None, "x"),
        check_vma=False,
    )
    out = fwd(hs, wq16, wk16, wv16, wo16, cos2, sin2, qw, kw)
    return out.reshape(B, S, HIDDEN)


# speedup vs baseline: 3.9116x; 3.9116x over previous
import jax
import jax.numpy as jnp
from jax.experimental import pallas as pl
from jax.experimental.pallas import tpu as pltpu

B = 1
S = 2048
HIDDEN = 2048
NH = 16
NKV = 8
G = NH // NKV
HD = 128
EPS = 1e-6
SCALE = HD ** -0.5

BP = 256
BQ = 512
BK = 512
BQ2 = BQ * G


def _rope(x, cos, sin):
    x1 = x[:, : HD // 2]
    x2 = x[:, HD // 2:]
    rot = jnp.concatenate([-x2, x1], axis=1)
    return x * cos + rot * sin


def _qkv_kernel(hs_ref, wq_ref, wk_ref, wv_ref, cos_ref, sin_ref,
                qw_ref, kw_ref, q_out, k_out, v_out):
    x = hs_ref[...].astype(jnp.bfloat16)
    cos = cos_ref[...]
    sin = sin_ref[...]
    qw = qw_ref[...]
    kw = kw_ref[...]

    q = jnp.dot(x, wq_ref[...], preferred_element_type=jnp.float32)
    for h in range(NH):
        qh = q[:, h * HD:(h + 1) * HD]
        var = jnp.mean(qh * qh, axis=-1, keepdims=True)
        qh = qh * (jax.lax.rsqrt(var + EPS) * SCALE) * qw
        q_out[h] = _rope(qh, cos, sin).astype(jnp.bfloat16)

    k = jnp.dot(x, wk_ref[...], preferred_element_type=jnp.float32)
    for h in range(NKV):
        kh = k[:, h * HD:(h + 1) * HD]
        var = jnp.mean(kh * kh, axis=-1, keepdims=True)
        kh = kh * jax.lax.rsqrt(var + EPS) * kw
        k_out[h] = _rope(kh, cos, sin).astype(jnp.bfloat16)

    v = jnp.dot(x, wv_ref[...], preferred_element_type=jnp.float32)
    for h in range(NKV):
        v_out[h] = v[:, h * HD:(h + 1) * HD].astype(jnp.bfloat16)


def _attn_kernel(q_ref, k_ref, v_ref, wo_ref, out_ref,
                 attn_ref, acc_ref, l_ref):
    i = pl.program_id(0)

    row = jax.lax.broadcasted_iota(jnp.int32, (BQ2, BK), 0)
    col = jax.lax.broadcasted_iota(jnp.int32, (BQ2, BK), 1)
    diag_mask = col <= jax.lax.rem(row, BQ)

    for p_ in range(NKV):
        q2 = q_ref[G * p_:G * p_ + G].reshape(BQ2, HD)

        l_ref[...] = jnp.zeros((BQ2, 1), jnp.float32)
        acc_ref[...] = jnp.zeros((BQ2, HD), jnp.float32)

        def body(j, _):
            kj = k_ref[p_, pl.ds(j * BK, BK), :]
            vj = v_ref[p_, pl.ds(j * BK, BK), :]
            s = jax.lax.dot_general(
                q2, kj, (((1,), (1,)), ((), ())),
                preferred_element_type=jnp.float32)
            p = jnp.exp(s)
            l_ref[...] += jnp.sum(p, axis=1, keepdims=True)
            acc_ref[...] += jnp.dot(p.astype(jnp.bfloat16), vj,
                                    preferred_element_type=jnp.float32)
            return 0

        jax.lax.fori_loop(0, i, body, 0)

        kj = k_ref[p_, pl.ds(i * BK, BK), :]
        vj = v_ref[p_, pl.ds(i * BK, BK), :]
        s = jax.lax.dot_general(
            q2, kj, (((1,), (1,)), ((), ())),
            preferred_element_type=jnp.float32)
        p = jnp.where(diag_mask, jnp.exp(s), 0.0)
        l_ref[...] += jnp.sum(p, axis=1, keepdims=True)
        acc_ref[...] += jnp.dot(p.astype(jnp.bfloat16), vj,
                                preferred_element_type=jnp.float32)

        out2 = (acc_ref[...] / l_ref[...]).astype(jnp.bfloat16)
        for g in range(G):
            h = G * p_ + g
            attn_ref[:, h * HD:(h + 1) * HD] = out2[g * BQ:(g + 1) * BQ]

    out_ref[...] = jnp.dot(attn_ref[...], wo_ref[...],
                           preferred_element_type=jnp.float32)


@jax.jit
def kernel(hidden_states, cos, sin, Wq, Wk, Wv, Wo, q_norm_w, k_norm_w):
    hs = hidden_states.reshape(S, HIDDEN)
    cos2 = cos.reshape(S, HD)
    sin2 = sin.reshape(S, HD)
    qw = q_norm_w.reshape(1, HD)
    kw = k_norm_w.reshape(1, HD)
    wq16 = Wq.astype(jnp.bfloat16)
    wk16 = Wk.astype(jnp.bfloat16)
    wv16 = Wv.astype(jnp.bfloat16)
    wo16 = Wo.astype(jnp.bfloat16)

    q, k, v = pl.pallas_call(
        _qkv_kernel,
        grid=(S // BP,),
        in_specs=[
            pl.BlockSpec((BP, HIDDEN), lambda i: (i, 0)),
            pl.BlockSpec((HIDDEN, NH * HD), lambda i: (0, 0)),
            pl.BlockSpec((HIDDEN, NKV * HD), lambda i: (0, 0)),
            pl.BlockSpec((HIDDEN, NKV * HD), lambda i: (0, 0)),
            pl.BlockSpec((BP, HD), lambda i: (i, 0)),
            pl.BlockSpec((BP, HD), lambda i: (i, 0)),
            pl.BlockSpec((1, HD), lambda i: (0, 0)),
            pl.BlockSpec((1, HD), lambda i: (0, 0)),
        ],
        out_specs=[
            pl.BlockSpec((NH, BP, HD), lambda i: (0, i, 0)),
            pl.BlockSpec((NKV, BP, HD), lambda i: (0, i, 0)),
            pl.BlockSpec((NKV, BP, HD), lambda i: (0, i, 0)),
        ],
        out_shape=[
            jax.ShapeDtypeStruct((NH, S, HD), jnp.bfloat16),
            jax.ShapeDtypeStruct((NKV, S, HD), jnp.bfloat16),
            jax.ShapeDtypeStruct((NKV, S, HD), jnp.bfloat16),
        ],
    )(hs, wq16, wk16, wv16, cos2, sin2, qw, kw)

    out = pl.pallas_call(
        _attn_kernel,
        grid=(S // BQ,),
        in_specs=[
            pl.BlockSpec((NH, BQ, HD), lambda i: (0, i, 0)),
            pl.BlockSpec((NKV, S, HD), lambda i: (0, 0, 0)),
            pl.BlockSpec((NKV, S, HD), lambda i: (0, 0, 0)),
            pl.BlockSpec((NH * HD, HIDDEN), lambda i: (0, 0)),
        ],
        out_specs=pl.BlockSpec((BQ, HIDDEN), lambda i: (i, 0)),
        out_shape=jax.ShapeDtypeStruct((S, HIDDEN), jnp.float32),
        scratch_shapes=[
            pltpu.VMEM((BQ, NH * HD), jnp.bfloat16),
            pltpu.VMEM((BQ2, HD), jnp.float32),
            pltpu.VMEM((BQ2, 1), jnp.float32),
        ],
    )(q, k, v, wo16)

    return out.reshape(B, S, HIDDEN)


# speedup vs baseline: 4.9737x; 1.2715x over previous
import jax
import jax.numpy as jnp
from jax.experimental import pallas as pl
from jax.experimental.pallas import tpu as pltpu

B = 1
S = 2048
HIDDEN = 2048
NH = 16
NKV = 8
G = NH // NKV
HD = 128
EPS = 1e-6
SCALE = HD ** -0.5

BP = 256
BQ = 512
BK = 512
BQ2 = BQ * G


def _rope(x, cos, sin):
    x1 = x[:, : HD // 2]
    x2 = x[:, HD // 2:]
    rot = jnp.concatenate([-x2, x1], axis=1)
    return x * cos + rot * sin


def _qkv_kernel(hs_ref, wq_ref, wk_ref, wv_ref, cos_ref, sin_ref,
                qw_ref, kw_ref, q_out, k_out, v_out):
    x = hs_ref[...].astype(jnp.bfloat16)
    cos = cos_ref[...]
    sin = sin_ref[...]
    qw = qw_ref[...]
    kw = kw_ref[...]

    q = jnp.dot(x, wq_ref[...], preferred_element_type=jnp.float32)
    for h in range(NH):
        qh = q[:, h * HD:(h + 1) * HD]
        var = jnp.mean(qh * qh, axis=-1, keepdims=True)
        qh = qh * (jax.lax.rsqrt(var + EPS) * SCALE) * qw
        q_out[h] = _rope(qh, cos, sin).astype(jnp.bfloat16)

    k = jnp.dot(x, wk_ref[...], preferred_element_type=jnp.float32)
    for h in range(NKV):
        kh = k[:, h * HD:(h + 1) * HD]
        var = jnp.mean(kh * kh, axis=-1, keepdims=True)
        kh = kh * jax.lax.rsqrt(var + EPS) * kw
        k_out[h] = _rope(kh, cos, sin).astype(jnp.bfloat16)

    v = jnp.dot(x, wv_ref[...], preferred_element_type=jnp.float32)
    for h in range(NKV):
        v_out[h] = v[:, h * HD:(h + 1) * HD].astype(jnp.bfloat16)


def _attn_kernel(q_ref, k_ref, v_ref, wo_ref, out_ref,
                 attn_ref, acc_ref, l_ref):
    i = pl.program_id(0)

    row = jax.lax.broadcasted_iota(jnp.int32, (BQ2, BK), 0)
    col = jax.lax.broadcasted_iota(jnp.int32, (BQ2, BK), 1)
    diag_mask = col <= jax.lax.rem(row, BQ)

    l_ref[...] = jnp.zeros((NKV, BQ2, 1), jnp.float32)
    acc_ref[...] = jnp.zeros((NKV, BQ2, HD), jnp.float32)

    def _step(p_, j, masked):
        q2 = q_ref[G * p_:G * p_ + G].reshape(BQ2, HD)
        kj = k_ref[p_, pl.ds(j * BK, BK), :]
        vj = v_ref[p_, pl.ds(j * BK, BK), :]
        s = jax.lax.dot_general(
            q2, kj, (((1,), (1,)), ((), ())),
            preferred_element_type=jnp.float32)
        p = jnp.exp(s)
        if masked:
            p = jnp.where(diag_mask, p, 0.0)
        l_ref[p_] += jnp.sum(p, axis=1, keepdims=True)
        acc_ref[p_] += jnp.dot(p.astype(jnp.bfloat16), vj,
                               preferred_element_type=jnp.float32)

    def body(j, _):
        for p_ in range(NKV):
            _step(p_, j, masked=False)
        return 0

    jax.lax.fori_loop(0, i, body, 0)

    for p_ in range(NKV):
        _step(p_, i, masked=True)

    for p_ in range(NKV):
        out2 = (acc_ref[p_] / l_ref[p_]).astype(jnp.bfloat16)
        for g in range(G):
            h = G * p_ + g
            attn_ref[:, h * HD:(h + 1) * HD] = out2[g * BQ:(g + 1) * BQ]

    out_ref[...] = jnp.dot(attn_ref[...], wo_ref[...],
                           preferred_element_type=jnp.float32)


@jax.jit
def kernel(hidden_states, cos, sin, Wq, Wk, Wv, Wo, q_norm_w, k_norm_w):
    hs = hidden_states.reshape(S, HIDDEN)
    cos2 = cos.reshape(S, HD)
    sin2 = sin.reshape(S, HD)
    qw = q_norm_w.reshape(1, HD)
    kw = k_norm_w.reshape(1, HD)
    wq16 = Wq.astype(jnp.bfloat16)
    wk16 = Wk.astype(jnp.bfloat16)
    wv16 = Wv.astype(jnp.bfloat16)
    wo16 = Wo.astype(jnp.bfloat16)

    q, k, v = pl.pallas_call(
        _qkv_kernel,
        grid=(S // BP,),
        in_specs=[
            pl.BlockSpec((BP, HIDDEN), lambda i: (i, 0)),
            pl.BlockSpec((HIDDEN, NH * HD), lambda i: (0, 0)),
            pl.BlockSpec((HIDDEN, NKV * HD), lambda i: (0, 0)),
            pl.BlockSpec((HIDDEN, NKV * HD), lambda i: (0, 0)),
            pl.BlockSpec((BP, HD), lambda i: (i, 0)),
            pl.BlockSpec((BP, HD), lambda i: (i, 0)),
            pl.BlockSpec((1, HD), lambda i: (0, 0)),
            pl.BlockSpec((1, HD), lambda i: (0, 0)),
        ],
        out_specs=[
            pl.BlockSpec((NH, BP, HD), lambda i: (0, i, 0)),
            pl.BlockSpec((NKV, BP, HD), lambda i: (0, i, 0)),
            pl.BlockSpec((NKV, BP, HD), lambda i: (0, i, 0)),
        ],
        out_shape=[
            jax.ShapeDtypeStruct((NH, S, HD), jnp.bfloat16),
            jax.ShapeDtypeStruct((NKV, S, HD), jnp.bfloat16),
            jax.ShapeDtypeStruct((NKV, S, HD), jnp.bfloat16),
        ],
    )(hs, wq16, wk16, wv16, cos2, sin2, qw, kw)

    out = pl.pallas_call(
        _attn_kernel,
        grid=(S // BQ,),
        in_specs=[
            pl.BlockSpec((NH, BQ, HD), lambda i: (0, i, 0)),
            pl.BlockSpec((NKV, S, HD), lambda i: (0, 0, 0)),
            pl.BlockSpec((NKV, S, HD), lambda i: (0, 0, 0)),
            pl.BlockSpec((NH * HD, HIDDEN), lambda i: (0, 0)),
        ],
        out_specs=pl.BlockSpec((BQ, HIDDEN), lambda i: (i, 0)),
        out_shape=jax.ShapeDtypeStruct((S, HIDDEN), jnp.float32),
        scratch_shapes=[
            pltpu.VMEM((BQ, NH * HD), jnp.bfloat16),
            pltpu.VMEM((NKV, BQ2, HD), jnp.float32),
            pltpu.VMEM((NKV, BQ2, 1), jnp.float32),
        ],
    )(q, k, v, wo16)

    return out.reshape(B, S, HIDDEN)
